# baseline (speedup 1.0000x reference)
import jax
import jax.numpy as jnp
from jax import lax
from jax.experimental import pallas as pl
from jax.experimental.pallas import tpu as pltpu

B, E, H, W = 4, 32, 128, 128
N, T = 8, 2
P_HALF = H * W
P_TOT = T * P_HALF
SR, SC = 256, 128
LOG_P = 15


def _lroll(x, shift, axis):
    return pltpu.roll(x, shift % x.shape[axis], axis)


def _stats_errors_kernel(m1_ref, m2_ref, gt_ref, keys_ref):
    f1 = m1_ref[0]
    f2 = m2_ref[0]
    g = gt_ref[0]
    gf = g.astype(jnp.float32)
    g1 = gf[:, :P_HALF]
    g2 = gf[:, P_HALF:]
    sq1 = f1 * f1
    sq2 = f2 * f2

    cdims = (((1,), (1,)), ((), ()))
    s1 = (lax.dot_general(g1, f1, cdims, preferred_element_type=jnp.float32)
          + lax.dot_general(g2, f2, cdims, preferred_element_type=jnp.float32))
    s2 = (lax.dot_general(g1, sq1, cdims, preferred_element_type=jnp.float32)
          + lax.dot_general(g2, sq2, cdims, preferred_element_type=jnp.float32))
    cnt = jnp.sum(gf, axis=1, keepdims=True)
    safe_cnt = jnp.maximum(cnt, 1.0)
    center = s1 / safe_cnt
    bw = (s2 - 2.0 * center * s1 + center * center * cnt) / jnp.maximum(
        cnt - 1.0, 1.0)

    wc = -2.0 * center * bw
    const = jnp.sum(center * center * bw, axis=1, keepdims=True)

    mdims = (((1,), (0,)), ((), ()))
    for half, (f, sq, gh, gi) in enumerate((
            (f1, sq1, g1, g[:, :P_HALF]),
            (f2, sq2, g2, g[:, P_HALF:]))):
        d = (lax.dot_general(wc, f, mdims, preferred_element_type=jnp.float32)
             + lax.dot_general(bw, sq, mdims,
                               preferred_element_type=jnp.float32)
             + const)
        probs = jnp.exp(-0.5 * d)
        err = jnp.where(gh > 0.5, 2.0 - 2.0 * probs, 2.0 * probs)
        err = jnp.maximum(err, 0.0)
        key = lax.bitcast_convert_type(err, jnp.int32)
        key = (key & ~jnp.int32(1)) | gi
        keys_ref[0, :, half * P_HALF:(half + 1) * P_HALF] = key


def _sort_loss_kernel(keys_ref, out_ref):
    x = keys_ref[...]
    i_row = lax.broadcasted_iota(jnp.int32, (1, SR, SC), 1)
    i_lane = lax.broadcasted_iota(jnp.int32, (1, SR, SC), 2)

    ni = x.shape[0]
    for s in range(1, LOG_P + 1):
        bit_s = ((i_row >> s) if s < 8 else (i_lane >> (s - 8))) & 1
        for j in range(s - 1, -1, -1):
            if 3 <= j < 8:
                dr = 1 << j
                grp = SR // (2 * dr)
                y = x.reshape(ni, grp, 2, dr, SC)
                a = lax.slice_in_dim(y, 0, 1, axis=2)
                b = lax.slice_in_dim(y, 1, 2, axis=2)
                mn = jnp.minimum(a, b)
                mx = jnp.maximum(a, b)
                if s < 8:
                    u = (lax.broadcasted_iota(
                        jnp.int32, (1, grp, 1, 1, 1), 1) >> (s - j - 1)) & 1
                else:
                    u = (lax.broadcasted_iota(
                        jnp.int32, (1, 1, 1, 1, SC), 4) >> (s - 8)) & 1
                first = jnp.where(u == 0, mx, mn)
                second = jnp.where(u == 0, mn, mx)
                x = lax.concatenate([first, second], 2).reshape(ni, SR, SC)
                continue
            if j < 8:
                ax, dd, i_ax, jb = 1, 1 << j, i_row, j
            else:
                ax, dd, i_ax, jb = 2, 1 << (j - 8), i_lane, j - 8
            bit_j = (i_ax >> jb) & 1
            xp = jnp.where(bit_j == 0, _lroll(x, -dd, ax), _lroll(x, dd, ax))
            mn = jnp.minimum(x, xp)
            mx = jnp.maximum(x, xp)
            x = jnp.where(bit_s == bit_j, mx, mn)

    gs = (x & 1).astype(jnp.float32)
    es = lax.bitcast_convert_type(x & ~jnp.int32(1), jnp.float32)

    c = gs
    for t in range(8):
        k = 1 << t
        sh = jnp.where(i_row >= k, _lroll(c, k, 1), 0.0)
        c = c + sh
    coltot = lax.slice_in_dim(c, SR - 1, SR, axis=1)
    cc = coltot
    for t in range(7):
        k = 1 << t
        sh = jnp.where(i_lane[:, :1, :] >= k, _lroll(cc, k, 2), 0.0)
        cc = cc + sh
    c = c + (cc - coltot)

    p_tot = lax.slice(c, (0, SR - 1, SC - 1), (x.shape[0], SR, SC))
    pos = (i_lane * SR + i_row + 1).astype(jnp.float32)
    union = p_tot + (pos - c)
    inter = p_tot - c
    jac = 1.0 - inter / union
    a = _lroll(jac, 1, axis=1)
    b = _lroll(a, 1, axis=2)
    jprev = jnp.where(i_row == 0, b, a)
    jprev = jnp.where((i_lane == 0) & (i_row == 0), 0.0, jprev)
    li = jnp.sum(es * (jac - jprev), axis=(1, 2), keepdims=True)

    valid = (p_tot > 0.0).astype(jnp.float32)
    n_valid = jnp.sum(valid, axis=0)
    lsum = jnp.sum(li * valid, axis=0)
    out_ref[...] = jnp.where(n_valid == 0.0, 0.0,
                             lsum / jnp.maximum(n_valid, 1.0))


@jax.jit
def kernel(mask_feats_1, mask_feats_2, gt_final):
    m1 = mask_feats_1.reshape(B, E, P_HALF)
    m2 = mask_feats_2.reshape(B, E, P_HALF)
    gt = gt_final.reshape(B, N, P_TOT)

    keys = pl.pallas_call(
        _stats_errors_kernel,
        grid=(B,),
        in_specs=[
            pl.BlockSpec((1, E, P_HALF), lambda b: (b, 0, 0)),
            pl.BlockSpec((1, E, P_HALF), lambda b: (b, 0, 0)),
            pl.BlockSpec((1, N, P_TOT), lambda b: (b, 0, 0)),
        ],
        out_specs=pl.BlockSpec((1, N, P_TOT), lambda b: (b, 0, 0)),
        out_shape=jax.ShapeDtypeStruct((B, N, P_TOT), jnp.int32),
    )(m1, m2, gt)

    keys = keys.reshape(B * N, SR, SC)
    res = pl.pallas_call(
        _sort_loss_kernel,
        out_shape=jax.ShapeDtypeStruct((1, 1), jnp.float32),
    )(keys)
    return res.reshape(())


# speedup vs baseline: 1.0139x; 1.0139x over previous
import jax
import jax.numpy as jnp
from jax import lax
from jax.experimental import pallas as pl
from jax.experimental.pallas import tpu as pltpu

B, E, H, W = 4, 32, 128, 128
N, T = 8, 2
P_HALF = H * W
P_TOT = T * P_HALF
SR, SC = 256, 128
LOG_P = 15


def _roll(x, shift, axis):
    n = x.shape[axis]
    s = shift % n
    if s == 0:
        return x
    a = lax.slice_in_dim(x, n - s, n, axis=axis)
    b = lax.slice_in_dim(x, 0, n - s, axis=axis)
    return lax.concatenate([a, b], axis)


def _lroll(x, shift, axis):
    return pltpu.roll(x, shift % x.shape[axis], axis)


def _stats_errors_kernel(m1_ref, m2_ref, gt_ref, keys_ref):
    f1 = m1_ref[0]
    f2 = m2_ref[0]
    g = gt_ref[0]
    gf = g.astype(jnp.float32)
    g1 = gf[:, :P_HALF]
    g2 = gf[:, P_HALF:]
    sq1 = f1 * f1
    sq2 = f2 * f2

    cdims = (((1,), (1,)), ((), ()))
    s1 = (lax.dot_general(g1, f1, cdims, preferred_element_type=jnp.float32)
          + lax.dot_general(g2, f2, cdims, preferred_element_type=jnp.float32))
    s2 = (lax.dot_general(g1, sq1, cdims, preferred_element_type=jnp.float32)
          + lax.dot_general(g2, sq2, cdims, preferred_element_type=jnp.float32))
    cnt = jnp.sum(gf, axis=1, keepdims=True)
    safe_cnt = jnp.maximum(cnt, 1.0)
    center = s1 / safe_cnt
    bw = (s2 - 2.0 * center * s1 + center * center * cnt) / jnp.maximum(
        cnt - 1.0, 1.0)

    wc = -2.0 * center * bw
    const = jnp.sum(center * center * bw, axis=1, keepdims=True)

    mdims = (((1,), (0,)), ((), ()))
    for half, (f, sq, gh, gi) in enumerate((
            (f1, sq1, g1, g[:, :P_HALF]),
            (f2, sq2, g2, g[:, P_HALF:]))):
        d = (lax.dot_general(wc, f, mdims, preferred_element_type=jnp.float32)
             + lax.dot_general(bw, sq, mdims,
                               preferred_element_type=jnp.float32)
             + const)
        probs = jnp.exp(-0.5 * d)
        err = jnp.where(gh > 0.5, 2.0 - 2.0 * probs, 2.0 * probs)
        err = jnp.maximum(err, 0.0)
        key = lax.bitcast_convert_type(err, jnp.int32)
        key = (key & ~jnp.int32(1)) | gi
        keys_ref[0, :, half * P_HALF:(half + 1) * P_HALF] = key


def _sort_loss_kernel(keys_ref, out_ref):
    x = keys_ref[...]
    i_row = lax.broadcasted_iota(jnp.int32, (1, SR, SC), 1)
    i_lane = lax.broadcasted_iota(jnp.int32, (1, SR, SC), 2)

    ni = x.shape[0]
    for s in range(1, LOG_P + 1):
        bit_s = ((i_row >> s) if s < 8 else (i_lane >> (s - 8))) & 1
        for j in range(s - 1, -1, -1):
            if 3 <= j < 8:
                dr = 1 << j
                grp = SR // (2 * dr)
                y = x.reshape(ni, grp, 2, dr, SC)
                a = lax.slice_in_dim(y, 0, 1, axis=2)
                b = lax.slice_in_dim(y, 1, 2, axis=2)
                mn = jnp.minimum(a, b)
                mx = jnp.maximum(a, b)
                if s < 8:
                    u = (lax.broadcasted_iota(
                        jnp.int32, (1, grp, 1, 1, 1), 1) >> (s - j - 1)) & 1
                else:
                    u = (lax.broadcasted_iota(
                        jnp.int32, (1, 1, 1, 1, SC), 4) >> (s - 8)) & 1
                first = jnp.where(u == 0, mx, mn)
                second = jnp.where(u == 0, mn, mx)
                x = lax.concatenate([first, second], 2).reshape(ni, SR, SC)
                continue
            if j < 8:
                ax, dd, i_ax, jb = 1, 1 << j, i_row, j
                rollf = _roll
            else:
                ax, dd, i_ax, jb = 2, 1 << (j - 8), i_lane, j - 8
                rollf = _lroll
            bit_j = (i_ax >> jb) & 1
            xp = jnp.where(bit_j == 0, rollf(x, -dd, ax), rollf(x, dd, ax))
            mn = jnp.minimum(x, xp)
            mx = jnp.maximum(x, xp)
            x = jnp.where(bit_s == bit_j, mx, mn)

    gs = (x & 1).astype(jnp.float32)
    es = lax.bitcast_convert_type(x & ~jnp.int32(1), jnp.float32)

    c = gs
    for t in range(8):
        k = 1 << t
        sh = jnp.where(i_row >= k, _roll(c, k, 1), 0.0)
        c = c + sh
    coltot = lax.slice_in_dim(c, SR - 1, SR, axis=1)
    cc = coltot
    for t in range(7):
        k = 1 << t
        sh = jnp.where(i_lane[:, :1, :] >= k, _lroll(cc, k, 2), 0.0)
        cc = cc + sh
    c = c + (cc - coltot)

    p_tot = lax.slice(c, (0, SR - 1, SC - 1), (x.shape[0], SR, SC))
    pos = (i_lane * SR + i_row + 1).astype(jnp.float32)
    union = p_tot + (pos - c)
    inter = p_tot - c
    jac = 1.0 - inter / union
    a = _roll(jac, 1, axis=1)
    b = _lroll(a, 1, axis=2)
    jprev = jnp.where(i_row == 0, b, a)
    jprev = jnp.where((i_lane == 0) & (i_row == 0), 0.0, jprev)
    li = jnp.sum(es * (jac - jprev), axis=(1, 2), keepdims=True)

    valid = (p_tot > 0.0).astype(jnp.float32)
    n_valid = jnp.sum(valid, axis=0)
    lsum = jnp.sum(li * valid, axis=0)
    out_ref[...] = jnp.where(n_valid == 0.0, 0.0,
                             lsum / jnp.maximum(n_valid, 1.0))


@jax.jit
def kernel(mask_feats_1, mask_feats_2, gt_final):
    m1 = mask_feats_1.reshape(B, E, P_HALF)
    m2 = mask_feats_2.reshape(B, E, P_HALF)
    gt = gt_final.reshape(B, N, P_TOT)

    keys = pl.pallas_call(
        _stats_errors_kernel,
        grid=(B,),
        in_specs=[
            pl.BlockSpec((1, E, P_HALF), lambda b: (b, 0, 0)),
            pl.BlockSpec((1, E, P_HALF), lambda b: (b, 0, 0)),
            pl.BlockSpec((1, N, P_TOT), lambda b: (b, 0, 0)),
        ],
        out_specs=pl.BlockSpec((1, N, P_TOT), lambda b: (b, 0, 0)),
        out_shape=jax.ShapeDtypeStruct((B, N, P_TOT), jnp.int32),
    )(m1, m2, gt)

    keys = keys.reshape(B * N, SR, SC)
    res = pl.pallas_call(
        _sort_loss_kernel,
        out_shape=jax.ShapeDtypeStruct((1, 1), jnp.float32),
    )(keys)
    return res.reshape(())
